# baseline (speedup 1.0000x reference)
import jax
import jax.numpy as jnp
from jax import lax
from jax.experimental import pallas as pl
from jax.experimental.pallas import tpu as pltpu
from jax.experimental.pallas import tpu_sc as plsc

N_FACTORS = 32
BATCH = 16384
NC = 2
NS = 16
NW = NC * NS
L = 16
B_PER_W = BATCH // NW
CHUNK = 128
NCHUNK = B_PER_W // CHUNK


def _body(data_hbm, uf_hbm, vf_hbm, out_hbm,
          duo_v, uidx_v, iidx_v, urows_v, vrows_v, out_v, sem):
  wid = lax.axis_index("s") * NC + lax.axis_index("c")

  pltpu.sync_copy(data_hbm.at[pl.ds(wid * (2 * B_PER_W), 2 * B_PER_W)], duo_v)

  lanes = lax.iota(jnp.int32, L)
  idx_even = (lanes & 7) * 2
  idx_odd = idx_even + 1
  in_lo = lanes < 8

  def deint(g, carry):
    a = duo_v[pl.ds(g * 2 * L, L)]
    b = duo_v[pl.ds(g * 2 * L + L, L)]
    ua = jnp.take_along_axis(a, idx_even, axis=0, mode="promise_in_bounds")
    ub = jnp.take_along_axis(b, idx_even, axis=0, mode="promise_in_bounds")
    ia = jnp.take_along_axis(a, idx_odd, axis=0, mode="promise_in_bounds")
    ib = jnp.take_along_axis(b, idx_odd, axis=0, mode="promise_in_bounds")
    uidx_v[pl.ds(g * L, L)] = jnp.where(in_lo, ua, ub)
    iidx_v[pl.ds(g * L, L)] = jnp.where(in_lo, ia, ib)
    return carry

  lax.fori_loop(0, B_PER_W // L, deint, 0)

  copies = []
  for j in range(NCHUNK):
    copies.append(pltpu.async_copy(
        uf_hbm.at[uidx_v.at[pl.ds(j * CHUNK, CHUNK)]],
        urows_v.at[pl.ds(j * CHUNK, CHUNK)], sem))
    copies.append(pltpu.async_copy(
        vf_hbm.at[iidx_v.at[pl.ds(j * CHUNK, CHUNK)]],
        vrows_v.at[pl.ds(j * CHUNK, CHUNK)], sem))
  for c in copies:
    c.wait()

  def group(g, carry):
    acc = jnp.zeros((L,), jnp.float32)
    for j in range(L):
      r = g * L + j
      u0, u1 = plsc.unpack(urows_v[r, pl.ds(0, 2 * L)],
                           format=plsc.PackFormat.INTERLEAVED)
      v0, v1 = plsc.unpack(vrows_v[r, pl.ds(0, 2 * L)],
                           format=plsc.PackFormat.INTERLEAVED)
      s = u0 * v0 + u1 * v1
      acc = jnp.where(lanes == j, jnp.sum(s), acc)
    out_v[pl.ds(g * L, L)] = acc
    return carry

  lax.fori_loop(0, B_PER_W // L, group, 0)

  pltpu.sync_copy(out_v, out_hbm.at[pl.ds(wid * B_PER_W, B_PER_W)])


@jax.jit
def kernel(data, user_factors, item_factors):
  mesh = plsc.VectorSubcoreMesh(
      core_axis_name="c", subcore_axis_name="s", num_cores=NC,
      num_subcores=NS)
  run = pl.kernel(
      _body,
      out_type=jax.ShapeDtypeStruct((BATCH,), jnp.float32),
      mesh=mesh,
      compiler_params=pltpu.CompilerParams(
          needs_layout_passes=False, use_tc_tiling_on_sc=False),
      scratch_types=[
          pltpu.VMEM((2 * B_PER_W,), jnp.int32),
          pltpu.VMEM((B_PER_W,), jnp.int32),
          pltpu.VMEM((B_PER_W,), jnp.int32),
          pltpu.VMEM((B_PER_W, N_FACTORS), jnp.bfloat16),
          pltpu.VMEM((B_PER_W, N_FACTORS), jnp.bfloat16),
          pltpu.VMEM((B_PER_W,), jnp.float32),
          pltpu.SemaphoreType.DMA,
      ],
  )
  return run(data.reshape(2 * BATCH), user_factors.astype(jnp.bfloat16),
             item_factors.astype(jnp.bfloat16))


# speedup vs baseline: 1.1769x; 1.1769x over previous
import jax
import jax.numpy as jnp
from jax import lax
from jax.experimental import pallas as pl
from jax.experimental.pallas import tpu as pltpu
from jax.experimental.pallas import tpu_sc as plsc

N_FACTORS = 32
BATCH = 16384
NC = 2
NS = 16
NW = NC * NS
L = 16
B_PER_W = BATCH // NW
CHUNK = 128
NCHUNK = B_PER_W // CHUNK


def _body(data_hbm, uf_hbm, vf_hbm, out_hbm,
          duo_v, uidx_v, iidx_v, urows_v, vrows_v, out_v, sem):
  wid = lax.axis_index("s") * NC + lax.axis_index("c")

  pltpu.sync_copy(data_hbm.at[pl.ds(wid * (2 * B_PER_W), 2 * B_PER_W)], duo_v)

  lanes = lax.iota(jnp.int32, L)
  idx_even = (lanes & 7) * 2
  idx_odd = idx_even + 1
  in_lo = lanes < 8

  def deint(g, carry):
    a = duo_v[pl.ds(g * 2 * L, L)]
    b = duo_v[pl.ds(g * 2 * L + L, L)]
    ua = jnp.take_along_axis(a, idx_even, axis=0, mode="promise_in_bounds")
    ub = jnp.take_along_axis(b, idx_even, axis=0, mode="promise_in_bounds")
    ia = jnp.take_along_axis(a, idx_odd, axis=0, mode="promise_in_bounds")
    ib = jnp.take_along_axis(b, idx_odd, axis=0, mode="promise_in_bounds")
    uidx_v[pl.ds(g * L, L)] = jnp.where(in_lo, ua, ub)
    iidx_v[pl.ds(g * L, L)] = jnp.where(in_lo, ia, ib)
    return carry

  lax.fori_loop(0, B_PER_W // L, deint, 0)

  copies = []
  for j in range(NCHUNK):
    copies.append(pltpu.async_copy(
        uf_hbm.at[uidx_v.at[pl.ds(j * CHUNK, CHUNK)]],
        urows_v.at[pl.ds(j * CHUNK, CHUNK)], sem))
    copies.append(pltpu.async_copy(
        vf_hbm.at[iidx_v.at[pl.ds(j * CHUNK, CHUNK)]],
        vrows_v.at[pl.ds(j * CHUNK, CHUNK)], sem))
  for c in copies:
    c.wait()

  def group(g, carry):
    acc = jnp.zeros((L,), jnp.float32)
    for j in range(L):
      r = g * L + j
      u0 = urows_v[r, pl.ds(0, L)]
      u1 = urows_v[r, pl.ds(L, L)]
      v0 = vrows_v[r, pl.ds(0, L)]
      v1 = vrows_v[r, pl.ds(L, L)]
      s = u0 * v0 + u1 * v1
      acc = jnp.where(lanes == j, jnp.sum(s), acc)
    out_v[pl.ds(g * L, L)] = acc
    return carry

  lax.fori_loop(0, B_PER_W // L, group, 0)

  pltpu.sync_copy(out_v, out_hbm.at[pl.ds(wid * B_PER_W, B_PER_W)])


@jax.jit
def kernel(data, user_factors, item_factors):
  mesh = plsc.VectorSubcoreMesh(
      core_axis_name="c", subcore_axis_name="s", num_cores=NC,
      num_subcores=NS)
  run = pl.kernel(
      _body,
      out_type=jax.ShapeDtypeStruct((BATCH,), jnp.float32),
      mesh=mesh,
      compiler_params=pltpu.CompilerParams(
          needs_layout_passes=False, use_tc_tiling_on_sc=False),
      scratch_types=[
          pltpu.VMEM((2 * B_PER_W,), jnp.int32),
          pltpu.VMEM((B_PER_W,), jnp.int32),
          pltpu.VMEM((B_PER_W,), jnp.int32),
          pltpu.VMEM((B_PER_W, N_FACTORS), jnp.float32),
          pltpu.VMEM((B_PER_W, N_FACTORS), jnp.float32),
          pltpu.VMEM((B_PER_W,), jnp.float32),
          pltpu.SemaphoreType.DMA,
      ],
  )
  return run(data.reshape(2 * BATCH), user_factors, item_factors)
